# baseline (speedup 1.0000x reference)
import functools

import jax
import jax.numpy as jnp
from jax import lax
from jax.experimental import pallas as pl
from jax.experimental.pallas import tpu as pltpu
from jax.experimental.pallas import tpu_sc as plsc

D_FEAT = 256
D_EDGE = 16
D_HID = 512
D_OUT = 512
D_PACK = D_HID // 2

QMAX = 16383.0
BIAS = 16384

NC = 2
NS = 16
NW = NC * NS


def _pre_body(blk, x_ref, wa_ref, wb_ref, aq_ref, bq_ref, s_ref,
              a_scr, b_scr, m_scr):
    p = pl.program_id(0)
    i = pl.program_id(1)
    rows = pl.ds(i * blk, blk)

    @pl.when(p == 0)
    def _():
        xb = x_ref[...]
        ma = jnp.dot(xb, wa_ref[...], preferred_element_type=jnp.float32)
        mb = jnp.dot(xb, wb_ref[...], preferred_element_type=jnp.float32)
        a_scr[rows, :] = ma.astype(jnp.bfloat16)
        b_scr[rows, :] = mb.astype(jnp.bfloat16)
        bm = jnp.maximum(jnp.max(jnp.abs(ma)), jnp.max(jnp.abs(mb)))
        prev = jnp.where(i == 0, 0.0, m_scr[0])
        m_scr[0] = jnp.maximum(prev, bm)

    @pl.when(p == 1)
    def _():
        absmax = jnp.maximum(m_scr[0], 1e-30)
        inv = QMAX / absmax

        def q(m):
            qv = jnp.round(m.astype(jnp.float32) * inv).astype(jnp.int32) + BIAS
            return qv[:, :D_PACK] | (qv[:, D_PACK:] << 16)

        aq_ref[...] = q(a_scr[rows, :])
        bq_ref[...] = q(b_scr[rows, :])
        s_ref[...] = jnp.full((8, 128), absmax / QMAX, jnp.float32)


def _precompute_quant(x, w1s, w1d, blk):
    n = x.shape[0]
    grid = n // blk
    return pl.pallas_call(
        functools.partial(_pre_body, blk),
        grid=(2, grid),
        in_specs=[
            pl.BlockSpec((blk, D_FEAT), lambda p, i: (i * (1 - p), 0)),
            pl.BlockSpec((D_FEAT, D_HID), lambda p, i: (0, 0)),
            pl.BlockSpec((D_FEAT, D_HID), lambda p, i: (0, 0)),
        ],
        out_specs=[
            pl.BlockSpec((blk, D_PACK), lambda p, i: (i * p, 0)),
            pl.BlockSpec((blk, D_PACK), lambda p, i: (i * p, 0)),
            pl.BlockSpec((8, 128), lambda p, i: (0, 0)),
        ],
        out_shape=[
            jax.ShapeDtypeStruct((n, D_PACK), jnp.int32),
            jax.ShapeDtypeStruct((n, D_PACK), jnp.int32),
            jax.ShapeDtypeStruct((8, 128), jnp.float32),
        ],
        scratch_shapes=[
            pltpu.VMEM((n, D_HID), jnp.bfloat16),
            pltpu.VMEM((n, D_HID), jnp.bfloat16),
            pltpu.SMEM((1,), jnp.float32),
        ],
    )(x, w1s, w1d)


def _gather_body(sz, e_off, chunk, a_hbm, b_hbm, src_hbm, dst_hbm, g_hbm,
                 idx_s, idx_d, ba0, bb0, ba1, bb1,
                 sg_a, sg_b, sw0, sw1):
    e_per_w = sz // NW
    n_chunks = e_per_w // chunk
    wid = lax.axis_index("s") * NC + lax.axis_index("c")
    base = wid * e_per_w

    pltpu.sync_copy(src_hbm.at[pl.ds(e_off + base, e_per_w)], idx_s)
    pltpu.sync_copy(dst_hbm.at[pl.ds(e_off + base, e_per_w)], idx_d)

    bufs = ((ba0, bb0, sw0), (ba1, bb1, sw1))

    def issue_gathers(j, buf_set):
        ba, bb, _ = buf_set
        isl = pl.ds(j * chunk, chunk)
        pltpu.async_copy(a_hbm.at[idx_s.at[isl]], ba, sg_a)
        pltpu.async_copy(b_hbm.at[idx_d.at[isl]], bb, sg_b)

    def wait_gathers(buf_set):
        ba, bb, _ = buf_set
        pltpu.make_async_copy(a_hbm.at[idx_s.at[pl.ds(0, chunk)]],
                              ba, sg_a).wait()
        pltpu.make_async_copy(b_hbm.at[idx_d.at[pl.ds(0, chunk)]],
                              bb, sg_b).wait()

    def drain_write(buf_set):
        ba, _, sw = buf_set
        pltpu.make_async_copy(ba, g_hbm.at[pl.ds(0, chunk)], sw).wait()

    def add_and_write(j, buf_set):
        ba, bb, sw = buf_set
        def add_row(r, c2):
            for k in range(D_PACK // 16):
                sl = pl.ds(k * 16, 16)
                ba[r, sl] = ba[r, sl] + bb[r, sl]
            return c2
        lax.fori_loop(0, chunk, add_row, 0)
        pltpu.async_copy(ba, g_hbm.at[pl.ds(base + j * chunk, chunk)], sw)

    issue_gathers(0, bufs[0])

    def pair(i, carry):
        for parity in (0, 1):
            j = 2 * i + parity
            cur, nxt = bufs[parity], bufs[1 - parity]
            wait_gathers(cur)
            @pl.when(j > 0)
            def _():
                drain_write(nxt)
            @pl.when(j + 1 < n_chunks)
            def _():
                issue_gathers(j + 1, nxt)
            add_and_write(j, cur)
        return carry

    lax.fori_loop(0, n_chunks // 2, pair, 0)
    if n_chunks % 2:
        j = n_chunks - 1
        cur, nxt = bufs[j % 2], bufs[1 - j % 2]
        wait_gathers(cur)
        drain_write(nxt)
        add_and_write(j, cur)
        drain_write(cur)
    else:
        drain_write(bufs[1])


def _gather(a_q, b_q, src, dst, sz, e_off, chunk):
    e_per_w = sz // NW
    mesh = plsc.VectorSubcoreMesh(core_axis_name="c", subcore_axis_name="s")
    body = functools.partial(_gather_body, sz, e_off, chunk)
    return pl.kernel(
        body,
        out_type=jax.ShapeDtypeStruct((sz, D_PACK), jnp.int32),
        mesh=mesh,
        scratch_types=[
            pltpu.VMEM((e_per_w,), jnp.int32),
            pltpu.VMEM((e_per_w,), jnp.int32),
            pltpu.VMEM((chunk, D_PACK), jnp.int32),
            pltpu.VMEM((chunk, D_PACK), jnp.int32),
            pltpu.VMEM((chunk, D_PACK), jnp.int32),
            pltpu.VMEM((chunk, D_PACK), jnp.int32),
            pltpu.SemaphoreType.DMA,
            pltpu.SemaphoreType.DMA,
            pltpu.SemaphoreType.DMA,
            pltpu.SemaphoreType.DMA,
        ],
    )(a_q, b_q, src, dst)


def _mlp_body_carry(carry_ref, g_ref, s_ref, eat_ref, w1e_lo_ref,
                    w1e_hi_ref, b1_lo_ref, b1_hi_ref, w2_lo_ref, w2_hi_ref,
                    b2_ref, o_ref):
    del carry_ref
    _mlp_body(g_ref, s_ref, eat_ref, w1e_lo_ref, w1e_hi_ref, b1_lo_ref,
              b1_hi_ref, w2_lo_ref, w2_hi_ref, b2_ref, o_ref)


def _mlp_body(g_ref, s_ref, eat_ref, w1e_lo_ref, w1e_hi_ref,
              b1_lo_ref, b1_hi_ref, w2_lo_ref, w2_hi_ref, b2_ref, o_ref):
    gq = g_ref[...]
    s = s_ref[0, 0]
    g_lo = (gq & 0xFFFF).astype(jnp.float32) * s
    g_hi = ((gq >> 16) & 0xFFFF).astype(jnp.float32) * s
    ea_t = eat_ref[...]
    dn = (((0,), (0,)), ((), ()))
    pre_lo = g_lo + lax.dot_general(
        ea_t, w1e_lo_ref[...], dn, preferred_element_type=jnp.float32)
    pre_hi = g_hi + lax.dot_general(
        ea_t, w1e_hi_ref[...], dn, preferred_element_type=jnp.float32)
    h_lo = jnp.maximum(pre_lo + b1_lo_ref[...], 0.0).astype(jnp.bfloat16)
    h_hi = jnp.maximum(pre_hi + b1_hi_ref[...], 0.0).astype(jnp.bfloat16)
    acc = jnp.dot(h_lo, w2_lo_ref[...], preferred_element_type=jnp.float32)
    acc += jnp.dot(h_hi, w2_hi_ref[...], preferred_element_type=jnp.float32)
    o_ref[...] = acc + b2_ref[...]


def _mlp_slice(carry, g, s, ea_t, w1e, b1_lo, b1_hi, w2b, b2, blk,
               n_edges, base_rows):
    slice_edges = g.shape[0]
    grid = slice_edges // blk
    base = base_rows // blk
    in_specs = [
        pl.BlockSpec((blk, D_PACK), lambda i: (i, 0)),
        pl.BlockSpec((1, 1), lambda i: (0, 0)),
        pl.BlockSpec((D_EDGE, blk), lambda i: (0, i + base)),
        pl.BlockSpec((D_EDGE, D_PACK), lambda i: (0, 0)),
        pl.BlockSpec((D_EDGE, D_PACK), lambda i: (0, 0)),
        pl.BlockSpec((1, D_PACK), lambda i: (0, 0)),
        pl.BlockSpec((1, D_PACK), lambda i: (0, 0)),
        pl.BlockSpec((D_PACK, D_OUT), lambda i: (0, 0)),
        pl.BlockSpec((D_PACK, D_OUT), lambda i: (0, 0)),
        pl.BlockSpec((1, D_OUT), lambda i: (0, 0)),
    ]
    args = [g, s, ea_t, w1e[:, :D_PACK], w1e[:, D_PACK:],
            b1_lo, b1_hi, w2b[:D_PACK], w2b[D_PACK:], b2.reshape(1, -1)]
    if carry is None:
        body = _mlp_body
        kwargs = {}
    else:
        body = _mlp_body_carry
        in_specs = [pl.BlockSpec(memory_space=pl.ANY)] + in_specs
        args = [carry] + args
        kwargs = {"input_output_aliases": {0: 0}}
    return pl.pallas_call(
        body,
        grid=(grid,),
        in_specs=in_specs,
        out_specs=pl.BlockSpec((blk, D_OUT), lambda i: (i + base, 0)),
        out_shape=jax.ShapeDtypeStruct((n_edges, D_OUT), jnp.float32),
        **kwargs,
    )(*args)


def kernel(x, edge_index, edge_attr, W1, b1, W2, b2):
    src = edge_index[0].astype(jnp.int32)
    dst = edge_index[1].astype(jnp.int32)
    w1s = W1[:D_FEAT]
    w1d = W1[D_FEAT:2 * D_FEAT]
    w1e = W1[2 * D_FEAT:]
    ea_t = edge_attr.T
    w2b = W2.astype(jnp.bfloat16)

    a_q, b_q, s_out = _precompute_quant(x, w1s, w1d, blk=2000)
    scale = lax.slice(s_out, (0, 0), (1, 1))

    bias_c = 2.0 * BIAS * scale
    b1_lo = b1[:D_PACK].reshape(1, -1) - bias_c
    b1_hi = b1[D_PACK:].reshape(1, -1) - bias_c

    n_edges = src.shape[0]
    sizes = (12800, 38400, 38400, 38400, 25600, 6400)
    out = None
    off = 0
    for sz in sizes:
        g = _gather(a_q, b_q, src, dst, sz, off, chunk=40)
        out = _mlp_slice(out, g, scale, ea_t, w1e, b1_lo, b1_hi, w2b, b2,
                         blk=3200, n_edges=n_edges, base_rows=off)
        off += sz
    return out


# speedup vs baseline: 1.0058x; 1.0058x over previous
import functools

import jax
import jax.numpy as jnp
from jax import lax
from jax.experimental import pallas as pl
from jax.experimental.pallas import tpu as pltpu
from jax.experimental.pallas import tpu_sc as plsc

D_FEAT = 256
D_EDGE = 16
D_HID = 512
D_OUT = 512
D_PACK = D_HID // 2

QMAX = 16383.0
BIAS = 16384

NC = 2
NS = 16
NW = NC * NS


def _pre_body(blk, x_ref, wa_ref, wb_ref, aq_ref, bq_ref, s_ref,
              a_scr, b_scr, m_scr):
    p = pl.program_id(0)
    i = pl.program_id(1)
    rows = pl.ds(i * blk, blk)

    @pl.when(p == 0)
    def _():
        xb = x_ref[...]
        ma = jnp.dot(xb, wa_ref[...], preferred_element_type=jnp.float32)
        mb = jnp.dot(xb, wb_ref[...], preferred_element_type=jnp.float32)
        a_scr[rows, :] = ma.astype(jnp.bfloat16)
        b_scr[rows, :] = mb.astype(jnp.bfloat16)
        bm = jnp.maximum(jnp.max(jnp.abs(ma)), jnp.max(jnp.abs(mb)))
        prev = jnp.where(i == 0, 0.0, m_scr[0])
        m_scr[0] = jnp.maximum(prev, bm)

    @pl.when(p == 1)
    def _():
        absmax = jnp.maximum(m_scr[0], 1e-30)
        inv = QMAX / absmax

        def q(m):
            qv = jnp.round(m.astype(jnp.float32) * inv).astype(jnp.int32) + BIAS
            return qv[:, :D_PACK] | (qv[:, D_PACK:] << 16)

        aq_ref[...] = q(a_scr[rows, :])
        bq_ref[...] = q(b_scr[rows, :])
        s_ref[...] = jnp.full((8, 128), absmax / QMAX, jnp.float32)


def _precompute_quant(x, w1s, w1d, blk):
    n = x.shape[0]
    grid = n // blk
    return pl.pallas_call(
        functools.partial(_pre_body, blk),
        grid=(2, grid),
        in_specs=[
            pl.BlockSpec((blk, D_FEAT), lambda p, i: (i * (1 - p), 0)),
            pl.BlockSpec((D_FEAT, D_HID), lambda p, i: (0, 0)),
            pl.BlockSpec((D_FEAT, D_HID), lambda p, i: (0, 0)),
        ],
        out_specs=[
            pl.BlockSpec((blk, D_PACK), lambda p, i: (i * p, 0)),
            pl.BlockSpec((blk, D_PACK), lambda p, i: (i * p, 0)),
            pl.BlockSpec((8, 128), lambda p, i: (0, 0)),
        ],
        out_shape=[
            jax.ShapeDtypeStruct((n, D_PACK), jnp.int32),
            jax.ShapeDtypeStruct((n, D_PACK), jnp.int32),
            jax.ShapeDtypeStruct((8, 128), jnp.float32),
        ],
        scratch_shapes=[
            pltpu.VMEM((n, D_HID), jnp.bfloat16),
            pltpu.VMEM((n, D_HID), jnp.bfloat16),
            pltpu.SMEM((1,), jnp.float32),
        ],
    )(x, w1s, w1d)


def _gather_body(sz, e_off, chunk, a_hbm, b_hbm, src_hbm, dst_hbm, g_hbm,
                 idx_s, idx_d, ba0, bb0, ba1, bb1,
                 sg_a, sg_b, sw0, sw1):
    e_per_w = sz // NW
    n_chunks = e_per_w // chunk
    wid = lax.axis_index("s") * NC + lax.axis_index("c")
    base = wid * e_per_w

    pltpu.sync_copy(src_hbm.at[pl.ds(e_off + base, e_per_w)], idx_s)
    pltpu.sync_copy(dst_hbm.at[pl.ds(e_off + base, e_per_w)], idx_d)

    bufs = ((ba0, bb0, sw0), (ba1, bb1, sw1))

    def issue_gathers(j, buf_set):
        ba, bb, _ = buf_set
        isl = pl.ds(j * chunk, chunk)
        pltpu.async_copy(a_hbm.at[idx_s.at[isl]], ba, sg_a)
        pltpu.async_copy(b_hbm.at[idx_d.at[isl]], bb, sg_b)

    def wait_gathers(buf_set):
        ba, bb, _ = buf_set
        pltpu.make_async_copy(a_hbm.at[idx_s.at[pl.ds(0, chunk)]],
                              ba, sg_a).wait()
        pltpu.make_async_copy(b_hbm.at[idx_d.at[pl.ds(0, chunk)]],
                              bb, sg_b).wait()

    def drain_write(buf_set):
        ba, _, sw = buf_set
        pltpu.make_async_copy(ba, g_hbm.at[pl.ds(0, chunk)], sw).wait()

    def add_and_write(j, buf_set):
        ba, bb, sw = buf_set
        def add_row(r, c2):
            for k in range(D_PACK // 16):
                sl = pl.ds(k * 16, 16)
                ba[r, sl] = ba[r, sl] + bb[r, sl]
            return c2
        lax.fori_loop(0, chunk, add_row, 0)
        pltpu.async_copy(ba, g_hbm.at[pl.ds(base + j * chunk, chunk)], sw)

    issue_gathers(0, bufs[0])

    def pair(i, carry):
        for parity in (0, 1):
            j = 2 * i + parity
            cur, nxt = bufs[parity], bufs[1 - parity]
            wait_gathers(cur)
            @pl.when(j > 0)
            def _():
                drain_write(nxt)
            @pl.when(j + 1 < n_chunks)
            def _():
                issue_gathers(j + 1, nxt)
            add_and_write(j, cur)
        return carry

    lax.fori_loop(0, n_chunks // 2, pair, 0)
    if n_chunks % 2:
        j = n_chunks - 1
        cur, nxt = bufs[j % 2], bufs[1 - j % 2]
        wait_gathers(cur)
        drain_write(nxt)
        add_and_write(j, cur)
        drain_write(cur)
    else:
        drain_write(bufs[1])


def _gather(a_q, b_q, src, dst, sz, e_off, chunk):
    e_per_w = sz // NW
    mesh = plsc.VectorSubcoreMesh(core_axis_name="c", subcore_axis_name="s")
    body = functools.partial(_gather_body, sz, e_off, chunk)
    return pl.kernel(
        body,
        out_type=jax.ShapeDtypeStruct((sz, D_PACK), jnp.int32),
        mesh=mesh,
        scratch_types=[
            pltpu.VMEM((e_per_w,), jnp.int32),
            pltpu.VMEM((e_per_w,), jnp.int32),
            pltpu.VMEM((chunk, D_PACK), jnp.int32),
            pltpu.VMEM((chunk, D_PACK), jnp.int32),
            pltpu.VMEM((chunk, D_PACK), jnp.int32),
            pltpu.VMEM((chunk, D_PACK), jnp.int32),
            pltpu.SemaphoreType.DMA,
            pltpu.SemaphoreType.DMA,
            pltpu.SemaphoreType.DMA,
            pltpu.SemaphoreType.DMA,
        ],
    )(a_q, b_q, src, dst)


def _mlp_body_carry(carry_ref, g_ref, s_ref, eat_ref, w1e_lo_ref,
                    w1e_hi_ref, b1_lo_ref, b1_hi_ref, w2_lo_ref, w2_hi_ref,
                    b2_ref, o_ref):
    del carry_ref
    _mlp_body(g_ref, s_ref, eat_ref, w1e_lo_ref, w1e_hi_ref, b1_lo_ref,
              b1_hi_ref, w2_lo_ref, w2_hi_ref, b2_ref, o_ref)


def _mlp_body(g_ref, s_ref, eat_ref, w1e_lo_ref, w1e_hi_ref,
              b1_lo_ref, b1_hi_ref, w2_lo_ref, w2_hi_ref, b2_ref, o_ref):
    gq = g_ref[...]
    s = s_ref[0, 0]
    g_lo = (gq & 0xFFFF).astype(jnp.float32) * s
    g_hi = ((gq >> 16) & 0xFFFF).astype(jnp.float32) * s
    ea_t = eat_ref[...]
    dn = (((0,), (0,)), ((), ()))
    pre_lo = g_lo + lax.dot_general(
        ea_t, w1e_lo_ref[...], dn, preferred_element_type=jnp.float32)
    pre_hi = g_hi + lax.dot_general(
        ea_t, w1e_hi_ref[...], dn, preferred_element_type=jnp.float32)
    h_lo = jnp.maximum(pre_lo + b1_lo_ref[...], 0.0).astype(jnp.bfloat16)
    h_hi = jnp.maximum(pre_hi + b1_hi_ref[...], 0.0).astype(jnp.bfloat16)
    acc = jnp.dot(h_lo, w2_lo_ref[...], preferred_element_type=jnp.float32)
    acc += jnp.dot(h_hi, w2_hi_ref[...], preferred_element_type=jnp.float32)
    o_ref[...] = acc + b2_ref[...]


def _mlp_slice(carry, g, s, ea_t, w1e, b1_lo, b1_hi, w2b, b2, blk,
               n_edges, base_rows):
    slice_edges = g.shape[0]
    grid = slice_edges // blk
    base = base_rows // blk
    in_specs = [
        pl.BlockSpec((blk, D_PACK), lambda i: (i, 0)),
        pl.BlockSpec((1, 1), lambda i: (0, 0)),
        pl.BlockSpec((D_EDGE, blk), lambda i: (0, i + base)),
        pl.BlockSpec((D_EDGE, D_PACK), lambda i: (0, 0)),
        pl.BlockSpec((D_EDGE, D_PACK), lambda i: (0, 0)),
        pl.BlockSpec((1, D_PACK), lambda i: (0, 0)),
        pl.BlockSpec((1, D_PACK), lambda i: (0, 0)),
        pl.BlockSpec((D_PACK, D_OUT), lambda i: (0, 0)),
        pl.BlockSpec((D_PACK, D_OUT), lambda i: (0, 0)),
        pl.BlockSpec((1, D_OUT), lambda i: (0, 0)),
    ]
    args = [g, s, ea_t, w1e[:, :D_PACK], w1e[:, D_PACK:],
            b1_lo, b1_hi, w2b[:D_PACK], w2b[D_PACK:], b2.reshape(1, -1)]
    if carry is None:
        body = _mlp_body
        kwargs = {}
    else:
        body = _mlp_body_carry
        in_specs = [pl.BlockSpec(memory_space=pl.ANY)] + in_specs
        args = [carry] + args
        kwargs = {"input_output_aliases": {0: 0}}
    return pl.pallas_call(
        body,
        grid=(grid,),
        in_specs=in_specs,
        out_specs=pl.BlockSpec((blk, D_OUT), lambda i: (i + base, 0)),
        out_shape=jax.ShapeDtypeStruct((n_edges, D_OUT), jnp.float32),
        **kwargs,
    )(*args)


def kernel(x, edge_index, edge_attr, W1, b1, W2, b2):
    src = edge_index[0].astype(jnp.int32)
    dst = edge_index[1].astype(jnp.int32)
    w1s = W1[:D_FEAT]
    w1d = W1[D_FEAT:2 * D_FEAT]
    w1e = W1[2 * D_FEAT:]
    ea_t = edge_attr.T
    w2b = W2.astype(jnp.bfloat16)

    a_q, b_q, s_out = _precompute_quant(x, w1s, w1d, blk=2000)
    scale = lax.slice(s_out, (0, 0), (1, 1))

    bias_c = 2.0 * BIAS * scale
    b1_lo = b1[:D_PACK].reshape(1, -1) - bias_c
    b1_hi = b1[D_PACK:].reshape(1, -1) - bias_c

    n_edges = src.shape[0]
    sizes = (12800, 38400, 38400, 38400, 32000)
    out = None
    off = 0
    for sz in sizes:
        g = _gather(a_q, b_q, src, dst, sz, off, chunk=40)
        out = _mlp_slice(out, g, scale, ea_t, w1e, b1_lo, b1_hi, w2b, b2,
                         blk=3200, n_edges=n_edges, base_rows=off)
        off += sz
    return out


# speedup vs baseline: 1.0185x; 1.0127x over previous
import functools

import jax
import jax.numpy as jnp
from jax import lax
from jax.experimental import pallas as pl
from jax.experimental.pallas import tpu as pltpu
from jax.experimental.pallas import tpu_sc as plsc

D_FEAT = 256
D_EDGE = 16
D_HID = 512
D_OUT = 512
D_PACK = D_HID // 2

QMAX = 16383.0
BIAS = 16384

NC = 2
NS = 16
NW = NC * NS


def _pre_body(blk, x_ref, wa_ref, wb_ref, aq_ref, bq_ref, s_ref,
              a_scr, b_scr, m_scr):
    p = pl.program_id(0)
    i = pl.program_id(1)
    rows = pl.ds(i * blk, blk)

    @pl.when(p == 0)
    def _():
        xb = x_ref[...]
        ma = jnp.dot(xb, wa_ref[...], preferred_element_type=jnp.float32)
        mb = jnp.dot(xb, wb_ref[...], preferred_element_type=jnp.float32)
        a_scr[rows, :] = ma.astype(jnp.bfloat16)
        b_scr[rows, :] = mb.astype(jnp.bfloat16)
        bm = jnp.maximum(jnp.max(jnp.abs(ma)), jnp.max(jnp.abs(mb)))
        prev = jnp.where(i == 0, 0.0, m_scr[0])
        m_scr[0] = jnp.maximum(prev, bm)

    @pl.when(p == 1)
    def _():
        absmax = jnp.maximum(m_scr[0], 1e-30)
        inv = QMAX / absmax

        def q(m):
            qv = jnp.round(m.astype(jnp.float32) * inv).astype(jnp.int32) + BIAS
            return qv[:, :D_PACK] | (qv[:, D_PACK:] << 16)

        aq_ref[...] = q(a_scr[rows, :])
        bq_ref[...] = q(b_scr[rows, :])
        s_ref[...] = jnp.full((8, 128), absmax / QMAX, jnp.float32)


def _precompute_quant(x, w1s, w1d, blk):
    n = x.shape[0]
    grid = n // blk
    return pl.pallas_call(
        functools.partial(_pre_body, blk),
        grid=(2, grid),
        in_specs=[
            pl.BlockSpec((blk, D_FEAT), lambda p, i: (i * (1 - p), 0)),
            pl.BlockSpec((D_FEAT, D_HID), lambda p, i: (0, 0)),
            pl.BlockSpec((D_FEAT, D_HID), lambda p, i: (0, 0)),
        ],
        out_specs=[
            pl.BlockSpec((blk, D_PACK), lambda p, i: (i * p, 0)),
            pl.BlockSpec((blk, D_PACK), lambda p, i: (i * p, 0)),
            pl.BlockSpec((8, 128), lambda p, i: (0, 0)),
        ],
        out_shape=[
            jax.ShapeDtypeStruct((n, D_PACK), jnp.int32),
            jax.ShapeDtypeStruct((n, D_PACK), jnp.int32),
            jax.ShapeDtypeStruct((8, 128), jnp.float32),
        ],
        scratch_shapes=[
            pltpu.VMEM((n, D_HID), jnp.bfloat16),
            pltpu.VMEM((n, D_HID), jnp.bfloat16),
            pltpu.SMEM((1,), jnp.float32),
        ],
    )(x, w1s, w1d)


def _gather_body(sz, e_off, chunk, a_hbm, b_hbm, src_hbm, dst_hbm, g_hbm,
                 idx_s, idx_d, ba0, bb0, ba1, bb1,
                 sg_a, sg_b, sw0, sw1):
    e_per_w = sz // NW
    n_chunks = e_per_w // chunk
    wid = lax.axis_index("s") * NC + lax.axis_index("c")
    base = wid * e_per_w

    pltpu.sync_copy(src_hbm.at[pl.ds(e_off + base, e_per_w)], idx_s)
    pltpu.sync_copy(dst_hbm.at[pl.ds(e_off + base, e_per_w)], idx_d)

    bufs = ((ba0, bb0, sw0), (ba1, bb1, sw1))

    def issue_gathers(j, buf_set):
        ba, bb, _ = buf_set
        isl = pl.ds(j * chunk, chunk)
        pltpu.async_copy(a_hbm.at[idx_s.at[isl]], ba, sg_a)
        pltpu.async_copy(b_hbm.at[idx_d.at[isl]], bb, sg_b)

    def wait_gathers(buf_set):
        ba, bb, _ = buf_set
        pltpu.make_async_copy(a_hbm.at[idx_s.at[pl.ds(0, chunk)]],
                              ba, sg_a).wait()
        pltpu.make_async_copy(b_hbm.at[idx_d.at[pl.ds(0, chunk)]],
                              bb, sg_b).wait()

    def drain_write(buf_set):
        ba, _, sw = buf_set
        pltpu.make_async_copy(ba, g_hbm.at[pl.ds(0, chunk)], sw).wait()

    def add_and_write(j, buf_set):
        ba, bb, sw = buf_set
        def add_row(r, c2):
            for k in range(D_PACK // 16):
                sl = pl.ds(k * 16, 16)
                ba[r, sl] = ba[r, sl] + bb[r, sl]
            return c2
        lax.fori_loop(0, chunk, add_row, 0)
        pltpu.async_copy(ba, g_hbm.at[pl.ds(base + j * chunk, chunk)], sw)

    issue_gathers(0, bufs[0])

    def pair(i, carry):
        for parity in (0, 1):
            j = 2 * i + parity
            cur, nxt = bufs[parity], bufs[1 - parity]
            wait_gathers(cur)
            @pl.when(j > 0)
            def _():
                drain_write(nxt)
            @pl.when(j + 1 < n_chunks)
            def _():
                issue_gathers(j + 1, nxt)
            add_and_write(j, cur)
        return carry

    lax.fori_loop(0, n_chunks // 2, pair, 0)
    if n_chunks % 2:
        j = n_chunks - 1
        cur, nxt = bufs[j % 2], bufs[1 - j % 2]
        wait_gathers(cur)
        drain_write(nxt)
        add_and_write(j, cur)
        drain_write(cur)
    else:
        drain_write(bufs[1])


def _gather(a_q, b_q, src, dst, sz, e_off, chunk):
    e_per_w = sz // NW
    mesh = plsc.VectorSubcoreMesh(core_axis_name="c", subcore_axis_name="s")
    body = functools.partial(_gather_body, sz, e_off, chunk)
    return pl.kernel(
        body,
        out_type=jax.ShapeDtypeStruct((sz, D_PACK), jnp.int32),
        mesh=mesh,
        scratch_types=[
            pltpu.VMEM((e_per_w,), jnp.int32),
            pltpu.VMEM((e_per_w,), jnp.int32),
            pltpu.VMEM((chunk, D_PACK), jnp.int32),
            pltpu.VMEM((chunk, D_PACK), jnp.int32),
            pltpu.VMEM((chunk, D_PACK), jnp.int32),
            pltpu.VMEM((chunk, D_PACK), jnp.int32),
            pltpu.SemaphoreType.DMA,
            pltpu.SemaphoreType.DMA,
            pltpu.SemaphoreType.DMA,
            pltpu.SemaphoreType.DMA,
        ],
    )(a_q, b_q, src, dst)


def _mlp_body_carry(carry_ref, g_ref, s_ref, eat_ref, w1e_lo_ref,
                    w1e_hi_ref, b1_lo_ref, b1_hi_ref, w2_lo_ref, w2_hi_ref,
                    b2_ref, o_ref):
    del carry_ref
    _mlp_body(g_ref, s_ref, eat_ref, w1e_lo_ref, w1e_hi_ref, b1_lo_ref,
              b1_hi_ref, w2_lo_ref, w2_hi_ref, b2_ref, o_ref)


def _mlp_body(g_ref, s_ref, eat_ref, w1e_lo_ref, w1e_hi_ref,
              b1_lo_ref, b1_hi_ref, w2_lo_ref, w2_hi_ref, b2_ref, o_ref):
    gq = g_ref[...]
    s = s_ref[0, 0]
    g_lo = (gq & 0xFFFF).astype(jnp.float32) * s
    g_hi = ((gq >> 16) & 0xFFFF).astype(jnp.float32) * s
    ea_t = eat_ref[...]
    dn = (((0,), (0,)), ((), ()))
    pre_lo = g_lo + lax.dot_general(
        ea_t, w1e_lo_ref[...], dn, preferred_element_type=jnp.float32)
    pre_hi = g_hi + lax.dot_general(
        ea_t, w1e_hi_ref[...], dn, preferred_element_type=jnp.float32)
    h_lo = jnp.maximum(pre_lo + b1_lo_ref[...], 0.0).astype(jnp.bfloat16)
    h_hi = jnp.maximum(pre_hi + b1_hi_ref[...], 0.0).astype(jnp.bfloat16)
    acc = jnp.dot(h_lo, w2_lo_ref[...], preferred_element_type=jnp.float32)
    acc += jnp.dot(h_hi, w2_hi_ref[...], preferred_element_type=jnp.float32)
    o_ref[...] = acc + b2_ref[...]


def _mlp_slice(carry, g, s, ea_t, w1e, b1_lo, b1_hi, w2b, b2, blk,
               n_edges, base_rows):
    slice_edges = g.shape[0]
    grid = slice_edges // blk
    base = base_rows // blk
    in_specs = [
        pl.BlockSpec((blk, D_PACK), lambda i: (i, 0)),
        pl.BlockSpec((1, 1), lambda i: (0, 0)),
        pl.BlockSpec((D_EDGE, blk), lambda i: (0, i + base)),
        pl.BlockSpec((D_EDGE, D_PACK), lambda i: (0, 0)),
        pl.BlockSpec((D_EDGE, D_PACK), lambda i: (0, 0)),
        pl.BlockSpec((1, D_PACK), lambda i: (0, 0)),
        pl.BlockSpec((1, D_PACK), lambda i: (0, 0)),
        pl.BlockSpec((D_PACK, D_OUT), lambda i: (0, 0)),
        pl.BlockSpec((D_PACK, D_OUT), lambda i: (0, 0)),
        pl.BlockSpec((1, D_OUT), lambda i: (0, 0)),
    ]
    args = [g, s, ea_t, w1e[:, :D_PACK], w1e[:, D_PACK:],
            b1_lo, b1_hi, w2b[:D_PACK], w2b[D_PACK:], b2.reshape(1, -1)]
    if carry is None:
        body = _mlp_body
        kwargs = {}
    else:
        body = _mlp_body_carry
        in_specs = [pl.BlockSpec(memory_space=pl.ANY)] + in_specs
        args = [carry] + args
        kwargs = {"input_output_aliases": {0: 0}}
    return pl.pallas_call(
        body,
        grid=(grid,),
        in_specs=in_specs,
        out_specs=pl.BlockSpec((blk, D_OUT), lambda i: (i + base, 0)),
        out_shape=jax.ShapeDtypeStruct((n_edges, D_OUT), jnp.float32),
        **kwargs,
    )(*args)


def kernel(x, edge_index, edge_attr, W1, b1, W2, b2):
    src = edge_index[0].astype(jnp.int32)
    dst = edge_index[1].astype(jnp.int32)
    w1s = W1[:D_FEAT]
    w1d = W1[D_FEAT:2 * D_FEAT]
    w1e = W1[2 * D_FEAT:]
    ea_t = edge_attr.T
    w2b = W2.astype(jnp.bfloat16)

    a_q, b_q, s_out = _precompute_quant(x, w1s, w1d, blk=2000)
    scale = lax.slice(s_out, (0, 0), (1, 1))

    bias_c = 2.0 * BIAS * scale
    b1_lo = b1[:D_PACK].reshape(1, -1) - bias_c
    b1_hi = b1[D_PACK:].reshape(1, -1) - bias_c

    n_edges = src.shape[0]
    sizes = (12800, 38400, 38400, 38400, 32000)
    out = None
    off = 0
    for sz in sizes:
        g = _gather(a_q, b_q,
                    lax.slice(src, (off,), (off + sz,)),
                    lax.slice(dst, (off,), (off + sz,)),
                    sz, 0, chunk=40)
        out = _mlp_slice(out, g, scale, ea_t, w1e, b1_lo, b1_hi, w2b, b2,
                         blk=3200, n_edges=n_edges, base_rows=off)
        off += sz
    return out
